# Initial kernel scaffold; baseline (speedup 1.0000x reference)
import jax
import jax.numpy as jnp
from jax.experimental import pallas as pl


def kernel(t, x, edge_index, adj_vals, e):
    raise NotImplementedError("write your pallas kernel here")


# speedup vs baseline: 3.9423x; 3.9423x over previous
import jax
import jax.numpy as jnp
from jax import lax
from jax.experimental import pallas as pl
from jax.experimental.pallas import tpu as pltpu
from jax.experimental.pallas import tpu_sc as plsc

N_NODES = 10000
N_EDGES = 320000
D = 128

NC = 2
NS = 16
NW = NC * NS
B = 128
NCHUNK = 79
EPW = NCHUNK * B
E_PAD = NW * EPW
L = 16
SHIFT = 14
MASK = (1 << SHIFT) - 1

SUB_STRIDE = 624
SUB_SPAN = 640


def _sc_body(x_hbm, idx_hbm, vals_hbm, z_hbm, part_hbm,
             idxv, valsv, rows, srcb, dstb, acc, sem):
    c = lax.axis_index("c")
    s = lax.axis_index("s")
    wid = c * NS + s
    base_row = pl.multiple_of(s * SUB_STRIDE, 8)

    pltpu.sync_copy(z_hbm, acc.at[pl.ds(base_row, SUB_SPAN)])

    pltpu.sync_copy(idx_hbm.at[wid], idxv)
    pltpu.sync_copy(vals_hbm.at[wid], valsv)
    plsc.subcore_barrier()

    @pl.loop(0, NCHUNK)
    def _chunk(j):
        jvec = jnp.full((L,), j, jnp.int32)

        for g in range(B // L):
            pk = idxv[j, pl.ds(g * L, L)]
            srcb[pl.ds(g * L, L)] = pk & MASK
            dstb[pl.ds(g * L, L)] = pk >> SHIFT

        pltpu.async_copy(x_hbm.at[srcb], rows, sem).wait()

        @pl.loop(0, B, step=L)
        def _grp(base):
            for r in range(L):
                sp = plsc.load_gather(
                    valsv, [jvec, jnp.full((L,), base + r, jnp.int32)])
                for g in range(D // L):
                    sl = (base + r, pl.ds(g * L, L))
                    rows[sl] = rows[sl] * sp

        pltpu.sync_copy(rows, acc.at[dstb], add=True)

    plsc.subcore_barrier()
    pltpu.sync_copy(acc.at[pl.ds(base_row, SUB_SPAN)],
                    part_hbm.at[c, pl.ds(base_row, SUB_SPAN)])


@jax.jit
def _spmm_sc(x, idx3, vals3, zblk):
    mesh = plsc.VectorSubcoreMesh(core_axis_name="c", subcore_axis_name="s",
                                  num_cores=NC, num_subcores=NS)
    return pl.kernel(
        _sc_body,
        out_type=jax.ShapeDtypeStruct((NC, N_NODES, D), jnp.float32),
        mesh=mesh,
        scratch_types=[
            pltpu.VMEM((NCHUNK, B), jnp.int32),
            pltpu.VMEM((NCHUNK, B), jnp.float32),
            pltpu.VMEM((B, D), jnp.float32),
            pltpu.VMEM((B,), jnp.int32),
            pltpu.VMEM((B,), jnp.int32),
            pltpu.VMEM_SHARED((N_NODES, D), jnp.float32),
            pltpu.SemaphoreType.DMA,
        ],
        compiler_params=pltpu.CompilerParams(needs_layout_passes=False),
    )(x, idx3, vals3, zblk)


def _combine_body(p0_ref, p1_ref, e_ref, o_ref):
    o_ref[...] = p0_ref[...] + p1_ref[...] + e_ref[...]


@jax.jit
def _combine(p0, p1, e):
    grid = 10
    rows = N_NODES // grid
    spec = pl.BlockSpec((rows, D), lambda i: (i, 0))
    return pl.pallas_call(
        _combine_body,
        out_shape=jax.ShapeDtypeStruct((N_NODES, D), jnp.float32),
        grid=(grid,),
        in_specs=[spec, spec, spec],
        out_specs=spec,
    )(p0, p1, e)


def kernel(t, x, edge_index, adj_vals, e):
    src = edge_index[0].astype(jnp.int32)
    dst = edge_index[1].astype(jnp.int32)
    packed = src | (dst << SHIFT)
    pad = E_PAD - N_EDGES
    packed = jnp.concatenate([packed, jnp.zeros((pad,), jnp.int32)])
    vals = jnp.concatenate([adj_vals, jnp.zeros((pad,), jnp.float32)])
    idx3 = packed.reshape(NW, NCHUNK, B)
    vals3 = vals.reshape(NW, NCHUNK, B)
    zblk = jnp.zeros((SUB_SPAN, D), jnp.float32)
    parts = _spmm_sc(x, idx3, vals3, zblk)
    return _combine(parts[0], parts[1], e)
